# Initial kernel scaffold; baseline (speedup 1.0000x reference)
import jax
import jax.numpy as jnp
from jax.experimental import pallas as pl


def kernel(x, params, edge_index, batch):
    raise NotImplementedError("write your pallas kernel here")


# speedup vs baseline: 1.4628x; 1.4628x over previous
import functools

import jax
import jax.numpy as jnp
from jax import lax
from jax.experimental import pallas as pl
from jax.experimental.pallas import tpu as pltpu
from jax.experimental.pallas import tpu_sc as plsc

_N = 100000
_E = 1600000
_EPS = 1e-5

_BR = 2000
_G = _N // _BR

_NPAD = 100352
_BB = 1024
_NBU = _BB // 128
_E16P = 100352
_NBLK = _E16P // _BB
_EPAD = 16 * _E16P


def _make_sc_msgpass(w, nchunk):
    cpad = _NPAD // nchunk
    rpt = cpad // 16
    trash = cpad
    nhalf = nchunk // 2

    copies = []
    left = rpt
    while left > 0:
        copies.append(min(_BB, left))
        left -= _BB

    mesh = plsc.VectorSubcoreMesh(core_axis_name="c", subcore_axis_name="s")

    def body(h_ref, src_ref, dst_ref, out_ref, srcb, dstb, dlb, rows, acc, gsem):
        c = lax.axis_index("c")
        s = lax.axis_index("s")
        t0 = s * _E16P
        base = s * rpt
        for cc in range(nhalf):
            chunk = c + 2 * cc if nhalf > 1 else c
            lo = chunk * cpad
            lov = jnp.broadcast_to(lo, (16,)).astype(jnp.int32)

            def zrow(r, carry):
                for j in range(w // 16):
                    rows[r, pl.ds(j * 16, 16)] = jnp.zeros((16,), jnp.float32)
                return carry
            lax.fori_loop(0, _BB, zrow, 0)
            off = 0
            for sz in copies:
                pltpu.sync_copy(rows.at[pl.ds(0, sz)],
                                acc.at[pl.ds(base + off, sz)])
                off += sz
            plsc.subcore_barrier()

            def blk(b, carry):
                eb = t0 + b * _BB
                pltpu.sync_copy(src_ref.at[pl.ds(eb, _BB)], srcb)
                pltpu.sync_copy(dst_ref.at[pl.ds(eb, _BB)], dstb)
                cps = [
                    pltpu.async_copy(h_ref.at[srcb.at[pl.ds(i * 128, 128)]],
                                     rows.at[pl.ds(i * 128, 128)], gsem)
                    for i in range(_NBU)
                ]
                for i in range(_NBU):
                    for t in range(8):
                        d16 = dstb[pl.ds(i * 128 + t * 16, 16)]
                        dl = d16 - lov
                        m = (dl >= 0) & (dl < cpad)
                        dlb[i, pl.ds(t * 16, 16)] = jnp.where(m, dl, trash)
                for cp in cps:
                    cp.wait()
                for i in range(_NBU):
                    pltpu.sync_copy(rows.at[pl.ds(i * 128, 128)],
                                    acc.at[dlb.at[i]], add=True)
                return carry
            lax.fori_loop(0, _NBLK, blk, 0)
            plsc.subcore_barrier()

            off = 0
            for sz in copies:
                pltpu.sync_copy(acc.at[pl.ds(base + off, sz)],
                                out_ref.at[pl.ds(lo + base + off, sz)])
                off += sz

    return pl.kernel(
        body,
        out_type=jax.ShapeDtypeStruct((_NPAD, w), jnp.float32),
        mesh=mesh,
        compiler_params=pltpu.CompilerParams(use_tc_tiling_on_sc=False),
        scratch_types=[
            pltpu.VMEM((_BB,), jnp.int32),
            pltpu.VMEM((_BB,), jnp.int32),
            pltpu.VMEM((_NBU, 128), jnp.int32),
            pltpu.VMEM((_BB, w), jnp.float32),
            pltpu.VMEM_SHARED((cpad + 16, w), jnp.float32),
            pltpu.SemaphoreType.DMA,
        ],
    )


_sc_cache = {}


def _msgpass(h, src_p, dst_p):
    w = h.shape[1]
    if w not in _sc_cache:
        _sc_cache[w] = _make_sc_msgpass(w, 4 if w == 32 else 8)
    return _sc_cache[w](h, src_p, dst_p)


def _row_spec(f):
    return pl.BlockSpec((_BR, f), lambda i: (i, 0))


def _full_spec(r, f):
    return pl.BlockSpec((r, f), lambda i: (0, 0))


def _stat_out_spec(f):
    return pl.BlockSpec((1, 1, f), lambda i: (i, 0, 0))


def _stat_in_spec(f):
    return pl.BlockSpec((_G, 1, f), lambda i: (0, 0, 0))


def _emb_body(x_ref, w_ref, b_ref, z_ref, ps_ref, pq_ref):
    z = jnp.dot(x_ref[...], w_ref[...], preferred_element_type=jnp.float32)
    z = jnp.maximum(z + b_ref[...], 0.0)
    z_ref[...] = z
    ps_ref[0, 0, :] = z.sum(axis=0)
    pq_ref[0, 0, :] = (z * z).sum(axis=0)


def _emb(x8, w8, b):
    return pl.pallas_call(
        _emb_body,
        grid=(_G,),
        in_specs=[_row_spec(8), _full_spec(8, 32), _full_spec(1, 32)],
        out_specs=[_row_spec(32), _stat_out_spec(32), _stat_out_spec(32)],
        out_shape=[jax.ShapeDtypeStruct((_N, 32), jnp.float32),
                   jax.ShapeDtypeStruct((_G, 1, 32), jnp.float32),
                   jax.ShapeDtypeStruct((_G, 1, 32), jnp.float32)],
    )(x8, w8, b)


def _conv_body(agg_ref, h_ref, wr_ref, wo_ref, b_ref, z_ref, ps_ref, pq_ref):
    z = (jnp.dot(agg_ref[...], wr_ref[...], preferred_element_type=jnp.float32)
         + jnp.dot(h_ref[...], wo_ref[...], preferred_element_type=jnp.float32)
         + b_ref[...])
    z_ref[...] = z
    ps_ref[0, 0, :] = z.sum(axis=0)
    pq_ref[0, 0, :] = (z * z).sum(axis=0)


def _conv(agg, h, wr, wo, b):
    fi, fo = wr.shape
    return pl.pallas_call(
        _conv_body,
        grid=(_G,),
        in_specs=[_row_spec(fi), _row_spec(fi), _full_spec(fi, fo),
                  _full_spec(fi, fo), _full_spec(1, fo)],
        out_specs=[_row_spec(fo), _stat_out_spec(fo), _stat_out_spec(fo)],
        out_shape=[jax.ShapeDtypeStruct((_N, fo), jnp.float32),
                   jax.ShapeDtypeStruct((_G, 1, fo), jnp.float32),
                   jax.ShapeDtypeStruct((_G, 1, fo), jnp.float32)],
    )(agg, h, wr, wo, b)


def _conv_pre_body(aggr_ref, h_ref, wo_ref, b_ref, z_ref, ps_ref, pq_ref):
    z = (aggr_ref[...]
         + jnp.dot(h_ref[...], wo_ref[...], preferred_element_type=jnp.float32)
         + b_ref[...])
    z_ref[...] = z
    ps_ref[0, 0, :] = z.sum(axis=0)
    pq_ref[0, 0, :] = (z * z).sum(axis=0)


def _conv_pre(aggr, h, wo, b):
    fi, fo = wo.shape
    return pl.pallas_call(
        _conv_pre_body,
        grid=(_G,),
        in_specs=[_row_spec(fo), _row_spec(fi), _full_spec(fi, fo),
                  _full_spec(1, fo)],
        out_specs=[_row_spec(fo), _stat_out_spec(fo), _stat_out_spec(fo)],
        out_shape=[jax.ShapeDtypeStruct((_N, fo), jnp.float32),
                   jax.ShapeDtypeStruct((_G, 1, fo), jnp.float32),
                   jax.ShapeDtypeStruct((_G, 1, fo), jnp.float32)],
    )(aggr, h, wo, b)


def _bn_core(z, ps_ref, pq_ref, g_ref, b_ref):
    s = ps_ref[...].sum(axis=0)
    q = pq_ref[...].sum(axis=0)
    m = s / _N
    v = q / _N - m * m
    inv = g_ref[...] * lax.rsqrt(v + _EPS)
    return (z - m) * inv + b_ref[...]


def _bn_body(z_ref, ps_ref, pq_ref, g_ref, b_ref, o_ref, *, relu):
    o = _bn_core(z_ref[...], ps_ref, pq_ref, g_ref, b_ref)
    if relu:
        o = jnp.maximum(o, 0.0)
    o_ref[...] = o


def _bn(z, ps, pq, g, b, relu):
    f = z.shape[1]
    return pl.pallas_call(
        functools.partial(_bn_body, relu=relu),
        grid=(_G,),
        in_specs=[_row_spec(f), _stat_in_spec(f), _stat_in_spec(f),
                  _full_spec(1, f), _full_spec(1, f)],
        out_specs=_row_spec(f),
        out_shape=jax.ShapeDtypeStruct((_N, f), jnp.float32),
    )(z, ps, pq, g, b)


def _bn_mm_body(z_ref, ps_ref, pq_ref, g_ref, b_ref, wr_ref, h_ref, hr_ref):
    o = _bn_core(z_ref[...], ps_ref, pq_ref, g_ref, b_ref)
    o = jnp.maximum(o, 0.0)
    h_ref[...] = o
    hr_ref[...] = jnp.dot(o, wr_ref[...], preferred_element_type=jnp.float32)


def _bn_mm(z, ps, pq, g, b, wr):
    fi, fo = wr.shape
    return pl.pallas_call(
        _bn_mm_body,
        grid=(_G,),
        in_specs=[_row_spec(fi), _stat_in_spec(fi), _stat_in_spec(fi),
                  _full_spec(1, fi), _full_spec(1, fi), _full_spec(fi, fo)],
        out_specs=[_row_spec(fi), _row_spec(fo)],
        out_shape=[jax.ShapeDtypeStruct((_N, fi), jnp.float32),
                   jax.ShapeDtypeStruct((_N, fo), jnp.float32)],
    )(z, ps, pq, g, b, wr)


def _bn_head_body(z_ref, ps_ref, pq_ref, g_ref, b_ref, w1_ref, b1_ref,
                  w2_ref, b2_ref, o_ref):
    h = _bn_core(z_ref[...], ps_ref, pq_ref, g_ref, b_ref)
    h = jnp.maximum(h, 0.0)
    o1 = jnp.dot(h, w1_ref[...], preferred_element_type=jnp.float32)
    o1 = jnp.maximum(o1 + b1_ref[...], 0.0)
    o_ref[...] = (jnp.dot(o1, w2_ref[...], preferred_element_type=jnp.float32)
                  + b2_ref[...])


def _bn_head(z, ps, pq, g, b, w1, b1, w2, b2):
    return pl.pallas_call(
        _bn_head_body,
        grid=(_G,),
        in_specs=[_row_spec(32), _stat_in_spec(32), _stat_in_spec(32),
                  _full_spec(1, 32), _full_spec(1, 32), _full_spec(32, 16),
                  _full_spec(1, 16), _full_spec(16, 2), _full_spec(1, 2)],
        out_specs=_row_spec(2),
        out_shape=jax.ShapeDtypeStruct((_N, 2), jnp.float32),
    )(z, ps, pq, g, b, w1, b1, w2, b2)


def kernel(x, params, edge_index, batch):
    p = params
    src = edge_index[0]
    dst = edge_index[1]
    pad = _EPAD - _E
    src_p = jnp.concatenate([src, jnp.zeros((pad,), jnp.int32)])
    dst_p = jnp.concatenate([dst, jnp.full((pad,), -1, jnp.int32)])

    x8 = jnp.pad(x, ((0, 0), (0, 3)))
    w8 = jnp.pad(p['emb_W'], ((0, 3), (0, 0)))
    r1 = lambda a: a.reshape(1, -1)

    z0, ps, pq = _emb(x8, w8, r1(p['emb_b']))
    h0 = _bn(z0, ps, pq, r1(p['emb_g']), r1(p['emb_be']), relu=False)

    agg0 = _msgpass(h0, src_p, dst_p)
    z1, ps, pq = _conv(agg0, h0, p['rel_W0'], p['root_W0'], r1(p['rel_b0']))
    h1 = _bn(z1, ps, pq, r1(p['bn_g0']), r1(p['bn_b0']), relu=True)

    agg1 = _msgpass(h1, src_p, dst_p)
    z2, ps, pq = _conv(agg1, h1, p['rel_W1'], p['root_W1'], r1(p['rel_b1']))

    h2, hr2 = _bn_mm(z2, ps, pq, r1(p['bn_g1']), r1(p['bn_b1']), p['rel_W2'])
    agg2 = _msgpass(hr2, src_p, dst_p)
    z3, ps, pq = _conv_pre(agg2, h2, p['root_W2'], r1(p['rel_b2']))

    h3, hr3 = _bn_mm(z3, ps, pq, r1(p['bn_g2']), r1(p['bn_b2']), p['rel_W3'])
    agg3 = _msgpass(hr3, src_p, dst_p)
    z4, ps, pq = _conv_pre(agg3, h3, p['root_W3'], r1(p['rel_b3']))

    return _bn_head(z4, ps, pq, r1(p['bn_g3']), r1(p['bn_b3']),
                    p['o_W1'], r1(p['o_b1']), p['o_W2'], r1(p['o_b2']))


# speedup vs baseline: 5.7186x; 3.9094x over previous
import functools

import jax
import jax.numpy as jnp
from jax import lax
from jax.experimental import pallas as pl
from jax.experimental.pallas import tpu as pltpu
from jax.experimental.pallas import tpu_sc as plsc

_N = 100000
_E = 1600000
_EPS = 1e-5

_BR = 2000
_G = _N // _BR

_NPAD = 101376
_BB = 1024
_E16P = 100352
_NBLK = _E16P // _BB
_EPAD = 16 * _E16P
_SLOT = 272


def _make_sc_msgpass(w, nchunk):
    cpad = _NPAD // nchunk
    rpt = cpad // 16
    trash = cpad
    nhalf = nchunk // 2

    copies = []
    left = rpt
    while left > 0:
        copies.append(min(128, left))
        left -= 128

    mesh = plsc.VectorSubcoreMesh(core_axis_name="c", subcore_axis_name="s")

    def body(h_ref, src_ref, dst_ref, out_ref,
             stg_s, stg_d, sbuf, dbuf, didx, rows, acc, gsem, ssem):
        c = lax.axis_index("c")
        s = lax.axis_index("s")
        t0 = s * _E16P
        base = s * rpt
        zf16 = jnp.zeros((16,), jnp.float32)
        zi16 = jnp.zeros((16,), jnp.int32)
        tv16 = jnp.full((16,), trash, jnp.int32)

        def drain_and_scatter(slot):
            pltpu.make_async_copy(h_ref.at[pl.ds(0, 128)], rows, gsem).wait()
            for t in range(8):
                didx[0, pl.ds(t * 16, 16)] = dbuf[pl.ds(slot * _SLOT + t * 16, 16)]
            pltpu.sync_copy(rows, acc.at[didx.at[0]], add=True)

        for cc in range(nhalf):
            chunk = c + 2 * cc if nhalf > 1 else c
            lo = chunk * cpad
            lov = jnp.broadcast_to(lo, (16,)).astype(jnp.int32)

            def zrow(r, carry):
                for j in range(w // 16):
                    rows[r, pl.ds(j * 16, 16)] = zf16
                return carry
            lax.fori_loop(0, 128, zrow, 0)
            off = 0
            for sz in copies:
                pltpu.sync_copy(rows.at[pl.ds(0, sz)],
                                acc.at[pl.ds(base + off, sz)])
                off += sz
            plsc.subcore_barrier()

            pltpu.async_copy(src_ref.at[pl.ds(t0, _BB)],
                             stg_s.at[pl.ds(0, _BB)], ssem)
            pltpu.async_copy(dst_ref.at[pl.ds(t0, _BB)],
                             stg_d.at[pl.ds(0, _BB)], ssem)

            def blk(b, car):
                hb = lax.rem(b, 2)
                hoff = hb * _BB
                pltpu.make_async_copy(src_ref.at[pl.ds(t0, _BB)],
                                      stg_s.at[pl.ds(0, _BB)], ssem).wait()
                pltpu.make_async_copy(dst_ref.at[pl.ds(t0, _BB)],
                                      stg_d.at[pl.ds(0, _BB)], ssem).wait()

                @pl.when(b + 1 < _NBLK)
                def _():
                    nb = t0 + (b + 1) * _BB
                    noff = (1 - hb) * _BB
                    pltpu.async_copy(src_ref.at[pl.ds(nb, _BB)],
                                     stg_s.at[pl.ds(noff, _BB)], ssem)
                    pltpu.async_copy(dst_ref.at[pl.ds(nb, _BB)],
                                     stg_d.at[pl.ds(noff, _BB)], ssem)

                def grp(g, car2):
                    wo, jb, pend = car2
                    s16 = stg_s[pl.ds(hoff + g * 16, 16)]
                    d16 = stg_d[pl.ds(hoff + g * 16, 16)]
                    dl = d16 - lov
                    m = (dl >= 0) & (dl < cpad)
                    key = jnp.where(m, dl, trash)
                    ks, vs = plsc.sort_key_val(key, s16)
                    woff = jb * _SLOT + wo
                    sbuf[pl.ds(woff, 16)] = vs
                    dbuf[pl.ds(woff, 16)] = ks
                    cnt = plsc.all_reduce_population_count(m)[0]
                    wn = wo + cnt
                    fl = wn >= 128

                    @pl.when(fl)
                    def _():
                        @pl.when(pend > 0)
                        def _():
                            drain_and_scatter(1 - jb)
                        pltpu.async_copy(
                            h_ref.at[sbuf.at[pl.ds(jb * _SLOT, 128)]],
                            rows, gsem)
                        lv_s = sbuf[pl.ds(jb * _SLOT + 128, 16)]
                        lv_d = dbuf[pl.ds(jb * _SLOT + 128, 16)]
                        sbuf[pl.ds((1 - jb) * _SLOT, 16)] = lv_s
                        dbuf[pl.ds((1 - jb) * _SLOT, 16)] = lv_d

                    fli = fl.astype(jnp.int32)
                    return (wn - 128 * fli,
                            jnp.where(fl, 1 - jb, jb),
                            jnp.maximum(pend, fli))

                return lax.fori_loop(0, _BB // 16, grp, car)

            z = jnp.int32(0)
            wo, jb, pend = lax.fori_loop(0, _NBLK, blk, (z, z, z))

            woff = jb * _SLOT + wo
            for t in range(8):
                sbuf[pl.ds(woff + t * 16, 16)] = zi16
                dbuf[pl.ds(woff + t * 16, 16)] = tv16

            @pl.when(pend > 0)
            def _():
                drain_and_scatter(1 - jb)
            pltpu.async_copy(h_ref.at[sbuf.at[pl.ds(jb * _SLOT, 128)]],
                             rows, gsem)
            drain_and_scatter(jb)
            plsc.subcore_barrier()

            off = 0
            for sz in copies:
                pltpu.sync_copy(acc.at[pl.ds(base + off, sz)],
                                out_ref.at[pl.ds(lo + base + off, sz)])
                off += sz

    return pl.kernel(
        body,
        out_type=jax.ShapeDtypeStruct((_NPAD, w), jnp.float32),
        mesh=mesh,
        compiler_params=pltpu.CompilerParams(use_tc_tiling_on_sc=False,
                                             needs_layout_passes=False),
        scratch_types=[
            pltpu.VMEM((2 * _BB,), jnp.int32),
            pltpu.VMEM((2 * _BB,), jnp.int32),
            pltpu.VMEM((2 * _SLOT,), jnp.int32),
            pltpu.VMEM((2 * _SLOT,), jnp.int32),
            pltpu.VMEM((1, 128), jnp.int32),
            pltpu.VMEM((128, w), jnp.float32),
            pltpu.VMEM_SHARED((cpad + 16, w), jnp.float32),
            pltpu.SemaphoreType.DMA,
            pltpu.SemaphoreType.DMA,
        ],
    )


_sc_cache = {}


def _msgpass(h, src_p, dst_p):
    w = h.shape[1]
    if w not in _sc_cache:
        _sc_cache[w] = _make_sc_msgpass(w, 4 if w == 32 else 6)
    return _sc_cache[w](h, src_p, dst_p)


def _row_spec(f):
    return pl.BlockSpec((_BR, f), lambda i: (i, 0))


def _full_spec(r, f):
    return pl.BlockSpec((r, f), lambda i: (0, 0))


def _stat_out_spec(f):
    return pl.BlockSpec((1, 1, f), lambda i: (i, 0, 0))


def _stat_in_spec(f):
    return pl.BlockSpec((_G, 1, f), lambda i: (0, 0, 0))


def _emb_body(x_ref, w_ref, b_ref, z_ref, ps_ref, pq_ref):
    z = jnp.dot(x_ref[...], w_ref[...], preferred_element_type=jnp.float32)
    z = jnp.maximum(z + b_ref[...], 0.0)
    z_ref[...] = z
    mb = z.mean(axis=0)
    d = z - mb
    ps_ref[0, 0, :] = z.sum(axis=0)
    pq_ref[0, 0, :] = (d * d).sum(axis=0)


def _emb(x8, w8, b):
    return pl.pallas_call(
        _emb_body,
        grid=(_G,),
        in_specs=[_row_spec(8), _full_spec(8, 32), _full_spec(1, 32)],
        out_specs=[_row_spec(32), _stat_out_spec(32), _stat_out_spec(32)],
        out_shape=[jax.ShapeDtypeStruct((_N, 32), jnp.float32),
                   jax.ShapeDtypeStruct((_G, 1, 32), jnp.float32),
                   jax.ShapeDtypeStruct((_G, 1, 32), jnp.float32)],
    )(x8, w8, b)


def _conv_body(agg_ref, h_ref, wr_ref, wo_ref, b_ref, z_ref, ps_ref, pq_ref):
    z = (jnp.dot(agg_ref[...], wr_ref[...], preferred_element_type=jnp.float32)
         + jnp.dot(h_ref[...], wo_ref[...], preferred_element_type=jnp.float32)
         + b_ref[...])
    z_ref[...] = z
    mb = z.mean(axis=0)
    d = z - mb
    ps_ref[0, 0, :] = z.sum(axis=0)
    pq_ref[0, 0, :] = (d * d).sum(axis=0)


def _conv(agg, h, wr, wo, b):
    fi, fo = wr.shape
    return pl.pallas_call(
        _conv_body,
        grid=(_G,),
        in_specs=[_row_spec(fi), _row_spec(fi), _full_spec(fi, fo),
                  _full_spec(fi, fo), _full_spec(1, fo)],
        out_specs=[_row_spec(fo), _stat_out_spec(fo), _stat_out_spec(fo)],
        out_shape=[jax.ShapeDtypeStruct((_N, fo), jnp.float32),
                   jax.ShapeDtypeStruct((_G, 1, fo), jnp.float32),
                   jax.ShapeDtypeStruct((_G, 1, fo), jnp.float32)],
    )(agg, h, wr, wo, b)


def _conv_pre_body(aggr_ref, h_ref, wo_ref, b_ref, z_ref, ps_ref, pq_ref):
    z = (aggr_ref[...]
         + jnp.dot(h_ref[...], wo_ref[...], preferred_element_type=jnp.float32)
         + b_ref[...])
    z_ref[...] = z
    mb = z.mean(axis=0)
    d = z - mb
    ps_ref[0, 0, :] = z.sum(axis=0)
    pq_ref[0, 0, :] = (d * d).sum(axis=0)


def _conv_pre(aggr, h, wo, b):
    fi, fo = wo.shape
    return pl.pallas_call(
        _conv_pre_body,
        grid=(_G,),
        in_specs=[_row_spec(fo), _row_spec(fi), _full_spec(fi, fo),
                  _full_spec(1, fo)],
        out_specs=[_row_spec(fo), _stat_out_spec(fo), _stat_out_spec(fo)],
        out_shape=[jax.ShapeDtypeStruct((_N, fo), jnp.float32),
                   jax.ShapeDtypeStruct((_G, 1, fo), jnp.float32),
                   jax.ShapeDtypeStruct((_G, 1, fo), jnp.float32)],
    )(aggr, h, wo, b)


def _bn_core(z, ps_ref, pq_ref, g_ref, b_ref):
    ps = ps_ref[...]
    m = ps.sum(axis=0) / _N
    mb = ps / _BR
    dd = mb - m[None]
    v = (pq_ref[...].sum(axis=0) + _BR * (dd * dd).sum(axis=0)) / _N
    inv = g_ref[...] * lax.rsqrt(v + _EPS)
    return (z - m) * inv + b_ref[...]


def _bn_body(z_ref, ps_ref, pq_ref, g_ref, b_ref, o_ref, *, relu):
    o = _bn_core(z_ref[...], ps_ref, pq_ref, g_ref, b_ref)
    if relu:
        o = jnp.maximum(o, 0.0)
    o_ref[...] = o


def _bn(z, ps, pq, g, b, relu):
    f = z.shape[1]
    return pl.pallas_call(
        functools.partial(_bn_body, relu=relu),
        grid=(_G,),
        in_specs=[_row_spec(f), _stat_in_spec(f), _stat_in_spec(f),
                  _full_spec(1, f), _full_spec(1, f)],
        out_specs=_row_spec(f),
        out_shape=jax.ShapeDtypeStruct((_N, f), jnp.float32),
    )(z, ps, pq, g, b)


def _bn_mm_body(z_ref, ps_ref, pq_ref, g_ref, b_ref, wr_ref, h_ref, hr_ref):
    o = _bn_core(z_ref[...], ps_ref, pq_ref, g_ref, b_ref)
    o = jnp.maximum(o, 0.0)
    h_ref[...] = o
    hr_ref[...] = jnp.dot(o, wr_ref[...], preferred_element_type=jnp.float32)


def _bn_mm(z, ps, pq, g, b, wr):
    fi, fo = wr.shape
    return pl.pallas_call(
        _bn_mm_body,
        grid=(_G,),
        in_specs=[_row_spec(fi), _stat_in_spec(fi), _stat_in_spec(fi),
                  _full_spec(1, fi), _full_spec(1, fi), _full_spec(fi, fo)],
        out_specs=[_row_spec(fi), _row_spec(fo)],
        out_shape=[jax.ShapeDtypeStruct((_N, fi), jnp.float32),
                   jax.ShapeDtypeStruct((_N, fo), jnp.float32)],
    )(z, ps, pq, g, b, wr)


def _bn_head_body(z_ref, ps_ref, pq_ref, g_ref, b_ref, w1_ref, b1_ref,
                  w2_ref, b2_ref, o_ref):
    h = _bn_core(z_ref[...], ps_ref, pq_ref, g_ref, b_ref)
    h = jnp.maximum(h, 0.0)
    o1 = jnp.dot(h, w1_ref[...], preferred_element_type=jnp.float32)
    o1 = jnp.maximum(o1 + b1_ref[...], 0.0)
    o_ref[...] = (jnp.dot(o1, w2_ref[...], preferred_element_type=jnp.float32)
                  + b2_ref[...])


def _bn_head(z, ps, pq, g, b, w1, b1, w2, b2):
    return pl.pallas_call(
        _bn_head_body,
        grid=(_G,),
        in_specs=[_row_spec(32), _stat_in_spec(32), _stat_in_spec(32),
                  _full_spec(1, 32), _full_spec(1, 32), _full_spec(32, 16),
                  _full_spec(1, 16), _full_spec(16, 2), _full_spec(1, 2)],
        out_specs=_row_spec(2),
        out_shape=jax.ShapeDtypeStruct((_N, 2), jnp.float32),
    )(z, ps, pq, g, b, w1, b1, w2, b2)


def kernel(x, params, edge_index, batch):
    p = params
    src = edge_index[0]
    dst = edge_index[1]
    pad = _EPAD - _E
    src_p = jnp.concatenate([src, jnp.zeros((pad,), jnp.int32)])
    dst_p = jnp.concatenate([dst, jnp.full((pad,), -1, jnp.int32)])

    x8 = jnp.pad(x, ((0, 0), (0, 3)))
    w8 = jnp.pad(p['emb_W'], ((0, 3), (0, 0)))
    r1 = lambda a: a.reshape(1, -1)

    z0, ps, pq = _emb(x8, w8, r1(p['emb_b']))
    h0 = _bn(z0, ps, pq, r1(p['emb_g']), r1(p['emb_be']), relu=False)

    agg0 = _msgpass(h0, src_p, dst_p)
    z1, ps, pq = _conv(agg0, h0, p['rel_W0'], p['root_W0'], r1(p['rel_b0']))
    h1 = _bn(z1, ps, pq, r1(p['bn_g0']), r1(p['bn_b0']), relu=True)

    agg1 = _msgpass(h1, src_p, dst_p)
    z2, ps, pq = _conv(agg1, h1, p['rel_W1'], p['root_W1'], r1(p['rel_b1']))

    h2, hr2 = _bn_mm(z2, ps, pq, r1(p['bn_g1']), r1(p['bn_b1']), p['rel_W2'])
    agg2 = _msgpass(hr2, src_p, dst_p)
    z3, ps, pq = _conv_pre(agg2, h2, p['root_W2'], r1(p['rel_b2']))

    h3, hr3 = _bn_mm(z3, ps, pq, r1(p['bn_g2']), r1(p['bn_b2']), p['rel_W3'])
    agg3 = _msgpass(hr3, src_p, dst_p)
    z4, ps, pq = _conv_pre(agg3, h3, p['root_W3'], r1(p['rel_b3']))

    return _bn_head(z4, ps, pq, r1(p['bn_g3']), r1(p['bn_b3']),
                    p['o_W1'], r1(p['o_b1']), p['o_W2'], r1(p['o_b2']))


# speedup vs baseline: 5.8263x; 1.0188x over previous
import functools

import jax
import jax.numpy as jnp
from jax import lax
from jax.experimental import pallas as pl
from jax.experimental.pallas import tpu as pltpu
from jax.experimental.pallas import tpu_sc as plsc

_N = 100000
_E = 1600000
_EPS = 1e-5

_BR = 2000
_G = _N // _BR

_NPAD = 101376
_BB = 1024
_E16P = 100352
_NBLK = _E16P // _BB
_EPAD = 16 * _E16P
_SLOT = 272


def _make_sc_msgpass(w, nchunk):
    cpad = _NPAD // nchunk
    rpt = cpad // 16
    trash = cpad
    nhalf = nchunk // 2

    copies = []
    left = rpt
    while left > 0:
        copies.append(min(128, left))
        left -= 128

    mesh = plsc.VectorSubcoreMesh(core_axis_name="c", subcore_axis_name="s")

    def body(h_ref, src_ref, dst_ref, out_ref,
             stg_s, stg_d, sbuf, dbuf, didx, rows, acc, gsem, ssem):
        c = lax.axis_index("c")
        s = lax.axis_index("s")
        t0 = s * _E16P
        base = s * rpt
        zf16 = jnp.zeros((16,), jnp.float32)
        zi16 = jnp.zeros((16,), jnp.int32)
        tv16 = jnp.full((16,), trash, jnp.int32)

        def drain_and_scatter(slot):
            pltpu.make_async_copy(h_ref.at[pl.ds(0, 128)], rows, gsem).wait()
            for t in range(8):
                didx[0, pl.ds(t * 16, 16)] = dbuf[pl.ds(slot * _SLOT + t * 16, 16)]
            pltpu.sync_copy(rows, acc.at[didx.at[0]], add=True)

        for cc in range(nhalf):
            chunk = c + 2 * cc if nhalf > 1 else c
            lo = chunk * cpad
            lov = jnp.broadcast_to(lo, (16,)).astype(jnp.int32)

            def zrow(r, carry):
                for j in range(w // 16):
                    rows[r, pl.ds(j * 16, 16)] = zf16
                return carry
            lax.fori_loop(0, 128, zrow, 0)
            off = 0
            for sz in copies:
                pltpu.sync_copy(rows.at[pl.ds(0, sz)],
                                acc.at[pl.ds(base + off, sz)])
                off += sz
            plsc.subcore_barrier()

            pltpu.async_copy(src_ref.at[pl.ds(t0, _BB)],
                             stg_s.at[pl.ds(0, _BB)], ssem)
            pltpu.async_copy(dst_ref.at[pl.ds(t0, _BB)],
                             stg_d.at[pl.ds(0, _BB)], ssem)

            def blk(b, car):
                hb = lax.rem(b, 2)
                hoff = hb * _BB
                pltpu.make_async_copy(src_ref.at[pl.ds(t0, _BB)],
                                      stg_s.at[pl.ds(0, _BB)], ssem).wait()
                pltpu.make_async_copy(dst_ref.at[pl.ds(t0, _BB)],
                                      stg_d.at[pl.ds(0, _BB)], ssem).wait()

                @pl.when(b + 1 < _NBLK)
                def _():
                    nb = t0 + (b + 1) * _BB
                    noff = (1 - hb) * _BB
                    pltpu.async_copy(src_ref.at[pl.ds(nb, _BB)],
                                     stg_s.at[pl.ds(noff, _BB)], ssem)
                    pltpu.async_copy(dst_ref.at[pl.ds(nb, _BB)],
                                     stg_d.at[pl.ds(noff, _BB)], ssem)

                def grp(g, car2):
                    wo, jb, pend = car2
                    s16 = stg_s[pl.ds(hoff + g * 16, 16)]
                    d16 = stg_d[pl.ds(hoff + g * 16, 16)]
                    dl = d16 - lov
                    m = (dl >= 0) & (dl < cpad)
                    key = jnp.where(m, dl, trash)
                    ks, vs = plsc.sort_key_val(key, s16)
                    woff = jb * _SLOT + wo
                    sbuf[pl.ds(woff, 16)] = vs
                    dbuf[pl.ds(woff, 16)] = ks
                    cnt = plsc.all_reduce_population_count(m)[0]
                    wn = wo + cnt
                    fl = wn >= 128

                    @pl.when(fl)
                    def _():
                        @pl.when(pend > 0)
                        def _():
                            drain_and_scatter(1 - jb)
                        pltpu.async_copy(
                            h_ref.at[sbuf.at[pl.ds(jb * _SLOT, 128)]],
                            rows, gsem)
                        lv_s = sbuf[pl.ds(jb * _SLOT + 128, 16)]
                        lv_d = dbuf[pl.ds(jb * _SLOT + 128, 16)]
                        sbuf[pl.ds((1 - jb) * _SLOT, 16)] = lv_s
                        dbuf[pl.ds((1 - jb) * _SLOT, 16)] = lv_d

                    fli = fl.astype(jnp.int32)
                    return (wn - 128 * fli,
                            jnp.where(fl, 1 - jb, jb),
                            jnp.maximum(pend, fli))

                return lax.fori_loop(0, _BB // 16, grp, car, unroll=8)

            z = jnp.int32(0)
            wo, jb, pend = lax.fori_loop(0, _NBLK, blk, (z, z, z))

            woff = jb * _SLOT + wo
            for t in range(8):
                sbuf[pl.ds(woff + t * 16, 16)] = zi16
                dbuf[pl.ds(woff + t * 16, 16)] = tv16

            @pl.when(pend > 0)
            def _():
                drain_and_scatter(1 - jb)
            pltpu.async_copy(h_ref.at[sbuf.at[pl.ds(jb * _SLOT, 128)]],
                             rows, gsem)
            drain_and_scatter(jb)
            plsc.subcore_barrier()

            off = 0
            for sz in copies:
                pltpu.sync_copy(acc.at[pl.ds(base + off, sz)],
                                out_ref.at[pl.ds(lo + base + off, sz)])
                off += sz

    return pl.kernel(
        body,
        out_type=jax.ShapeDtypeStruct((_NPAD, w), jnp.float32),
        mesh=mesh,
        compiler_params=pltpu.CompilerParams(use_tc_tiling_on_sc=False,
                                             needs_layout_passes=False),
        scratch_types=[
            pltpu.VMEM((2 * _BB,), jnp.int32),
            pltpu.VMEM((2 * _BB,), jnp.int32),
            pltpu.VMEM((2 * _SLOT,), jnp.int32),
            pltpu.VMEM((2 * _SLOT,), jnp.int32),
            pltpu.VMEM((1, 128), jnp.int32),
            pltpu.VMEM((128, w), jnp.float32),
            pltpu.VMEM_SHARED((cpad + 16, w), jnp.float32),
            pltpu.SemaphoreType.DMA,
            pltpu.SemaphoreType.DMA,
        ],
    )


_sc_cache = {}


def _msgpass(h, src_p, dst_p):
    w = h.shape[1]
    if w not in _sc_cache:
        _sc_cache[w] = _make_sc_msgpass(w, 4 if w == 32 else 6)
    return _sc_cache[w](h, src_p, dst_p)


def _row_spec(f):
    return pl.BlockSpec((_BR, f), lambda i: (i, 0))


def _full_spec(r, f):
    return pl.BlockSpec((r, f), lambda i: (0, 0))


def _stat_out_spec(f):
    return pl.BlockSpec((1, 1, f), lambda i: (i, 0, 0))


def _stat_in_spec(f):
    return pl.BlockSpec((_G, 1, f), lambda i: (0, 0, 0))


def _emb_body(x_ref, w_ref, b_ref, z_ref, ps_ref, pq_ref):
    z = jnp.dot(x_ref[...], w_ref[...], preferred_element_type=jnp.float32)
    z = jnp.maximum(z + b_ref[...], 0.0)
    z_ref[...] = z
    mb = z.mean(axis=0)
    d = z - mb
    ps_ref[0, 0, :] = z.sum(axis=0)
    pq_ref[0, 0, :] = (d * d).sum(axis=0)


def _emb(x8, w8, b):
    return pl.pallas_call(
        _emb_body,
        grid=(_G,),
        in_specs=[_row_spec(8), _full_spec(8, 32), _full_spec(1, 32)],
        out_specs=[_row_spec(32), _stat_out_spec(32), _stat_out_spec(32)],
        out_shape=[jax.ShapeDtypeStruct((_N, 32), jnp.float32),
                   jax.ShapeDtypeStruct((_G, 1, 32), jnp.float32),
                   jax.ShapeDtypeStruct((_G, 1, 32), jnp.float32)],
    )(x8, w8, b)


def _conv_body(agg_ref, h_ref, wr_ref, wo_ref, b_ref, z_ref, ps_ref, pq_ref):
    z = (jnp.dot(agg_ref[...], wr_ref[...], preferred_element_type=jnp.float32)
         + jnp.dot(h_ref[...], wo_ref[...], preferred_element_type=jnp.float32)
         + b_ref[...])
    z_ref[...] = z
    mb = z.mean(axis=0)
    d = z - mb
    ps_ref[0, 0, :] = z.sum(axis=0)
    pq_ref[0, 0, :] = (d * d).sum(axis=0)


def _conv(agg, h, wr, wo, b):
    fi, fo = wr.shape
    return pl.pallas_call(
        _conv_body,
        grid=(_G,),
        in_specs=[_row_spec(fi), _row_spec(fi), _full_spec(fi, fo),
                  _full_spec(fi, fo), _full_spec(1, fo)],
        out_specs=[_row_spec(fo), _stat_out_spec(fo), _stat_out_spec(fo)],
        out_shape=[jax.ShapeDtypeStruct((_N, fo), jnp.float32),
                   jax.ShapeDtypeStruct((_G, 1, fo), jnp.float32),
                   jax.ShapeDtypeStruct((_G, 1, fo), jnp.float32)],
    )(agg, h, wr, wo, b)


def _conv_pre_body(aggr_ref, h_ref, wo_ref, b_ref, z_ref, ps_ref, pq_ref):
    z = (aggr_ref[...]
         + jnp.dot(h_ref[...], wo_ref[...], preferred_element_type=jnp.float32)
         + b_ref[...])
    z_ref[...] = z
    mb = z.mean(axis=0)
    d = z - mb
    ps_ref[0, 0, :] = z.sum(axis=0)
    pq_ref[0, 0, :] = (d * d).sum(axis=0)


def _conv_pre(aggr, h, wo, b):
    fi, fo = wo.shape
    return pl.pallas_call(
        _conv_pre_body,
        grid=(_G,),
        in_specs=[_row_spec(fo), _row_spec(fi), _full_spec(fi, fo),
                  _full_spec(1, fo)],
        out_specs=[_row_spec(fo), _stat_out_spec(fo), _stat_out_spec(fo)],
        out_shape=[jax.ShapeDtypeStruct((_N, fo), jnp.float32),
                   jax.ShapeDtypeStruct((_G, 1, fo), jnp.float32),
                   jax.ShapeDtypeStruct((_G, 1, fo), jnp.float32)],
    )(aggr, h, wo, b)


def _bn_core(z, ps_ref, pq_ref, g_ref, b_ref):
    ps = ps_ref[...]
    m = ps.sum(axis=0) / _N
    mb = ps / _BR
    dd = mb - m[None]
    v = (pq_ref[...].sum(axis=0) + _BR * (dd * dd).sum(axis=0)) / _N
    inv = g_ref[...] * lax.rsqrt(v + _EPS)
    return (z - m) * inv + b_ref[...]


def _bn_body(z_ref, ps_ref, pq_ref, g_ref, b_ref, o_ref, *, relu):
    o = _bn_core(z_ref[...], ps_ref, pq_ref, g_ref, b_ref)
    if relu:
        o = jnp.maximum(o, 0.0)
    o_ref[...] = o


def _bn(z, ps, pq, g, b, relu):
    f = z.shape[1]
    return pl.pallas_call(
        functools.partial(_bn_body, relu=relu),
        grid=(_G,),
        in_specs=[_row_spec(f), _stat_in_spec(f), _stat_in_spec(f),
                  _full_spec(1, f), _full_spec(1, f)],
        out_specs=_row_spec(f),
        out_shape=jax.ShapeDtypeStruct((_N, f), jnp.float32),
    )(z, ps, pq, g, b)


def _bn_mm_body(z_ref, ps_ref, pq_ref, g_ref, b_ref, wr_ref, h_ref, hr_ref):
    o = _bn_core(z_ref[...], ps_ref, pq_ref, g_ref, b_ref)
    o = jnp.maximum(o, 0.0)
    h_ref[...] = o
    hr_ref[...] = jnp.dot(o, wr_ref[...], preferred_element_type=jnp.float32)


def _bn_mm(z, ps, pq, g, b, wr):
    fi, fo = wr.shape
    return pl.pallas_call(
        _bn_mm_body,
        grid=(_G,),
        in_specs=[_row_spec(fi), _stat_in_spec(fi), _stat_in_spec(fi),
                  _full_spec(1, fi), _full_spec(1, fi), _full_spec(fi, fo)],
        out_specs=[_row_spec(fi), _row_spec(fo)],
        out_shape=[jax.ShapeDtypeStruct((_N, fi), jnp.float32),
                   jax.ShapeDtypeStruct((_N, fo), jnp.float32)],
    )(z, ps, pq, g, b, wr)


def _bn_head_body(z_ref, ps_ref, pq_ref, g_ref, b_ref, w1_ref, b1_ref,
                  w2_ref, b2_ref, o_ref):
    h = _bn_core(z_ref[...], ps_ref, pq_ref, g_ref, b_ref)
    h = jnp.maximum(h, 0.0)
    o1 = jnp.dot(h, w1_ref[...], preferred_element_type=jnp.float32)
    o1 = jnp.maximum(o1 + b1_ref[...], 0.0)
    o_ref[...] = (jnp.dot(o1, w2_ref[...], preferred_element_type=jnp.float32)
                  + b2_ref[...])


def _bn_head(z, ps, pq, g, b, w1, b1, w2, b2):
    return pl.pallas_call(
        _bn_head_body,
        grid=(_G,),
        in_specs=[_row_spec(32), _stat_in_spec(32), _stat_in_spec(32),
                  _full_spec(1, 32), _full_spec(1, 32), _full_spec(32, 16),
                  _full_spec(1, 16), _full_spec(16, 2), _full_spec(1, 2)],
        out_specs=_row_spec(2),
        out_shape=jax.ShapeDtypeStruct((_N, 2), jnp.float32),
    )(z, ps, pq, g, b, w1, b1, w2, b2)


def kernel(x, params, edge_index, batch):
    p = params
    src = edge_index[0]
    dst = edge_index[1]
    pad = _EPAD - _E
    src_p = jnp.concatenate([src, jnp.zeros((pad,), jnp.int32)])
    dst_p = jnp.concatenate([dst, jnp.full((pad,), -1, jnp.int32)])

    x8 = jnp.pad(x, ((0, 0), (0, 3)))
    w8 = jnp.pad(p['emb_W'], ((0, 3), (0, 0)))
    r1 = lambda a: a.reshape(1, -1)

    z0, ps, pq = _emb(x8, w8, r1(p['emb_b']))
    h0 = _bn(z0, ps, pq, r1(p['emb_g']), r1(p['emb_be']), relu=False)

    agg0 = _msgpass(h0, src_p, dst_p)
    z1, ps, pq = _conv(agg0, h0, p['rel_W0'], p['root_W0'], r1(p['rel_b0']))
    h1 = _bn(z1, ps, pq, r1(p['bn_g0']), r1(p['bn_b0']), relu=True)

    agg1 = _msgpass(h1, src_p, dst_p)
    z2, ps, pq = _conv(agg1, h1, p['rel_W1'], p['root_W1'], r1(p['rel_b1']))

    h2, hr2 = _bn_mm(z2, ps, pq, r1(p['bn_g1']), r1(p['bn_b1']), p['rel_W2'])
    agg2 = _msgpass(hr2, src_p, dst_p)
    z3, ps, pq = _conv_pre(agg2, h2, p['root_W2'], r1(p['rel_b2']))

    h3, hr3 = _bn_mm(z3, ps, pq, r1(p['bn_g2']), r1(p['bn_b2']), p['rel_W3'])
    agg3 = _msgpass(hr3, src_p, dst_p)
    z4, ps, pq = _conv_pre(agg3, h3, p['root_W3'], r1(p['rel_b3']))

    return _bn_head(z4, ps, pq, r1(p['bn_g3']), r1(p['bn_b3']),
                    p['o_W1'], r1(p['o_b1']), p['o_W2'], r1(p['o_b2']))
